# baseline (speedup 1.0000x reference)
import functools

import jax
import jax.numpy as jnp
from jax import lax
from jax.experimental import pallas as pl
from jax.experimental.pallas import tpu as pltpu
from jax.experimental.pallas import tpu_sc as plsc


def _emb_kernel(B, L, D, NW):
    CHUNKS = B // NW
    NSEG = L // 128
    SEG = 128
    LANES = 16

    mesh = plsc.VectorSubcoreMesh(core_axis_name="c", subcore_axis_name="s")

    @functools.partial(
        pl.kernel,
        out_type=jax.ShapeDtypeStruct((B, L, D), jnp.float32),
        mesh=mesh,
        compiler_params=pltpu.CompilerParams(use_tc_tiling_on_sc=False),
        scratch_types=[
            pltpu.VMEM((L, D), jnp.float32),
            pltpu.VMEM((2, NSEG, SEG), jnp.int32),
            pltpu.VMEM((2, L, D), jnp.float32),
            pltpu.SemaphoreType.DMA,
            pltpu.SemaphoreType.DMA,
        ],
    )
    def body(idx_hbm, table_hbm, pos_hbm, out_hbm, pos_v, idx_v, rows_v,
             sem0, sem1):
        sems = (sem0, sem1)
        wid = lax.axis_index("s") * 2 + lax.axis_index("c")
        base = wid * CHUNKS

        pltpu.sync_copy(pos_hbm, pos_v)

        def fire(slot, bid):
            pltpu.sync_copy(idx_hbm.at[bid], idx_v.at[slot])
            for j in range(NSEG):
                pltpu.async_copy(
                    table_hbm.at[idx_v.at[slot, j]],
                    rows_v.at[slot, pl.ds(j * SEG, SEG)],
                    sems[slot],
                )

        def process(slot, bid):
            for j in range(NSEG):
                pltpu.make_async_copy(
                    table_hbm.at[idx_v.at[slot, j]],
                    rows_v.at[slot, pl.ds(j * SEG, SEG)],
                    sems[slot],
                ).wait()
            rows_b = rows_v.at[slot]

            @plsc.parallel_loop(0, L, unroll=4)
            def _(i):
                for j in range(D // LANES):
                    sl = (i, pl.ds(j * LANES, LANES))
                    rows_b[sl] = rows_b[sl] + pos_v[sl]

            pltpu.sync_copy(rows_b, out_hbm.at[bid])

        fire(0, base)

        def chunk_pair(k, carry):
            c = base + k * 2
            fire(1, c + 1)
            process(0, c)

            @pl.when(k * 2 + 2 < CHUNKS)
            def _():
                fire(0, c + 2)

            process(1, c + 1)
            return carry

        lax.fori_loop(0, CHUNKS // 2, chunk_pair, jnp.int32(0))

    return body


def kernel(inputs, emb_table, pos_table):
    B, L = inputs.shape
    _, D = emb_table.shape
    NW = 32
    idx3 = inputs.reshape(B, L // 128, 128).astype(jnp.int32)
    return _emb_kernel(B, L, D, NW)(idx3, emb_table, pos_table)
